# baseline (speedup 1.0000x reference)
import numpy as np

import jax
import jax.numpy as jnp
from jax.experimental import pallas as pl

_C = 64
_K = 16
_DEPTH = 4
_NODES = _K - 1
_BN = 2048
_HALF = _BN // 2


def _halut_block_kernel(i_ref, a_ref, t_ref, tile_ref, kk_ref, l_ref, o_ref):
    f32 = jnp.float32
    ib = i_ref[...].astype(jnp.bfloat16)
    ia0 = jnp.dot(ib[:_HALF], a_ref[...], preferred_element_type=f32)
    ia1 = jnp.dot(ib[_HALF:], a_ref[...], preferred_element_type=f32)

    hs = []
    for lvl in range(_DEPTH):
        sl = slice(lvl * _C, (lvl + 1) * _C)
        h = jnp.concatenate([ia0[:, sl], ia1[:, sl]], axis=1)
        hs.append(h.astype(jnp.bfloat16).astype(f32))

    d = []
    for i in range(_NODES):
        lvl = (i + 1).bit_length() - 1
        di = jnp.tanh(hs[lvl] - t_ref[i, :][None, :])
        d.append(di.astype(jnp.bfloat16).astype(f32))

    p1 = [-d[0], d[0]]
    p2, p3, bs = [], [], []
    for j in range(4):
        b0, b1 = (j >> 1) & 1, j & 1
        n1 = 1 + b0
        p2.append(p1[b0] + (d[n1] if b1 else -d[n1]))
    for j in range(8):
        b0, b1, b2 = (j >> 2) & 1, (j >> 1) & 1, j & 1
        n2 = 3 + 2 * b0 + b1
        p3.append(p2[j >> 1] + (d[n2] if b2 else -d[n2]))
    for k in range(_K):
        b0, b1, b2 = (k >> 3) & 1, (k >> 2) & 1, (k >> 1) & 1
        n3 = 7 + 4 * b0 + 2 * b1 + b2
        bs.append(p3[k >> 1] + (d[n3] if (k & 1) else -d[n3]))

    best_v = bs[0]
    best_k = jnp.zeros(bs[0].shape, dtype=f32)
    for k in range(1, _K):
        upd = bs[k] > best_v
        best_v = jnp.where(upd, bs[k], best_v)
        best_k = jnp.where(upd, f32(k), best_k)

    codes = best_k.astype(jnp.bfloat16)

    for half, codes_h in ((0, codes[:, :_C]), (1, codes[:, _C:])):
        codes_t = jnp.dot(codes_h, tile_ref[...], preferred_element_type=f32)
        e = (codes_t == kk_ref[0:1, :]).astype(jnp.bfloat16)
        out = jax.lax.dot_general(
            e, l_ref[...], (((1,), (1,)), ((), ())),
            preferred_element_type=f32)
        o_ref[half * _HALF:(half + 1) * _HALF] = out


def kernel(I, T, L, S, B, A):
    del S, B
    n, dim = I.shape
    m = L.shape[0]
    a_perm = (A.reshape(dim, _C, _DEPTH).transpose(0, 2, 1)
              .reshape(dim, _C * _DEPTH).astype(jnp.bfloat16))
    t15 = T.reshape(_C, _NODES).T
    t_perm = jnp.pad(jnp.concatenate([t15, t15], axis=1), ((0, 1), (0, 0)))
    l4 = L.reshape(m, _C * _K).astype(jnp.bfloat16)
    tile = jnp.asarray(np.repeat(np.eye(_C, dtype=np.float32), _K, axis=1),
                       dtype=jnp.bfloat16)
    kk = jnp.asarray(np.broadcast_to(
        (np.arange(_K * _C) % _K).astype(np.float32), (8, _K * _C)))

    return pl.pallas_call(
        _halut_block_kernel,
        grid=(n // _BN,),
        in_specs=[
            pl.BlockSpec((_BN, dim), lambda i: (i, 0)),
            pl.BlockSpec((dim, _C * _DEPTH), lambda i: (0, 0)),
            pl.BlockSpec((_K, 2 * _C), lambda i: (0, 0)),
            pl.BlockSpec((_C, _K * _C), lambda i: (0, 0)),
            pl.BlockSpec((8, _K * _C), lambda i: (0, 0)),
            pl.BlockSpec((m, _K * _C), lambda i: (0, 0)),
        ],
        out_specs=pl.BlockSpec((_BN, m), lambda i: (i, 0)),
        out_shape=jax.ShapeDtypeStruct((n, m), jnp.float32),
    )(I, a_perm, t_perm, tile, kk, l4)


# speedup vs baseline: 1.0452x; 1.0452x over previous
import numpy as np

import jax
import jax.numpy as jnp
from jax.experimental import pallas as pl
from jax.experimental.pallas import tpu as pltpu

_C = 64
_K = 16
_DEPTH = 4
_NODES = _K - 1
_BN = 1024
_HALF = _BN // 2


def _halut_block_kernel(i_ref, a_ref, t_ref, tile_ref, kk_ref, l_ref, o_ref):
    f32 = jnp.float32
    ib = i_ref[...].astype(jnp.bfloat16)
    ia0 = jnp.dot(ib[:_HALF], a_ref[...], preferred_element_type=f32)
    ia1 = jnp.dot(ib[_HALF:], a_ref[...], preferred_element_type=f32)

    hs = []
    for lvl in range(_DEPTH):
        sl = slice(lvl * _C, (lvl + 1) * _C)
        h = jnp.concatenate([ia0[:, sl], ia1[:, sl]], axis=1)
        hs.append(h.astype(jnp.bfloat16).astype(f32))

    d = []
    for i in range(_NODES):
        lvl = (i + 1).bit_length() - 1
        di = jnp.tanh(hs[lvl] - t_ref[i, :][None, :])
        d.append(di.astype(jnp.bfloat16).astype(f32))

    p1 = [-d[0], d[0]]
    p2, p3, bs = [], [], []
    for j in range(4):
        b0, b1 = (j >> 1) & 1, j & 1
        n1 = 1 + b0
        p2.append(p1[b0] + (d[n1] if b1 else -d[n1]))
    for j in range(8):
        b0, b1, b2 = (j >> 2) & 1, (j >> 1) & 1, j & 1
        n2 = 3 + 2 * b0 + b1
        p3.append(p2[j >> 1] + (d[n2] if b2 else -d[n2]))
    for k in range(_K):
        b0, b1, b2 = (k >> 3) & 1, (k >> 2) & 1, (k >> 1) & 1
        n3 = 7 + 4 * b0 + 2 * b1 + b2
        bs.append(p3[k >> 1] + (d[n3] if (k & 1) else -d[n3]))

    best_v = bs[0]
    best_k = jnp.zeros(bs[0].shape, dtype=f32)
    for k in range(1, _K):
        upd = bs[k] > best_v
        best_v = jnp.where(upd, bs[k], best_v)
        best_k = jnp.where(upd, f32(k), best_k)

    codes = best_k.astype(jnp.bfloat16)

    for half, codes_h in ((0, codes[:, :_C]), (1, codes[:, _C:])):
        codes_t = jnp.dot(codes_h, tile_ref[...], preferred_element_type=f32)
        e = (codes_t == kk_ref[0:1, :]).astype(jnp.bfloat16)
        out = jax.lax.dot_general(
            e, l_ref[...], (((1,), (1,)), ((), ())),
            preferred_element_type=f32)
        o_ref[half * _HALF:(half + 1) * _HALF] = out


def kernel(I, T, L, S, B, A):
    del S, B
    n, dim = I.shape
    m = L.shape[0]
    a_perm = (A.reshape(dim, _C, _DEPTH).transpose(0, 2, 1)
              .reshape(dim, _C * _DEPTH).astype(jnp.bfloat16))
    t15 = T.reshape(_C, _NODES).T
    t_perm = jnp.pad(jnp.concatenate([t15, t15], axis=1), ((0, 1), (0, 0)))
    l4 = L.reshape(m, _C * _K).astype(jnp.bfloat16)
    tile = jnp.asarray(np.repeat(np.eye(_C, dtype=np.float32), _K, axis=1),
                       dtype=jnp.bfloat16)
    kk = jnp.asarray(np.broadcast_to(
        (np.arange(_K * _C) % _K).astype(np.float32), (8, _K * _C)))

    return pl.pallas_call(
        _halut_block_kernel,
        grid=(n // _BN,),
        in_specs=[
            pl.BlockSpec((_BN, dim), lambda i: (i, 0)),
            pl.BlockSpec((dim, _C * _DEPTH), lambda i: (0, 0)),
            pl.BlockSpec((_K, 2 * _C), lambda i: (0, 0)),
            pl.BlockSpec((_C, _K * _C), lambda i: (0, 0)),
            pl.BlockSpec((8, _K * _C), lambda i: (0, 0)),
            pl.BlockSpec((m, _K * _C), lambda i: (0, 0)),
        ],
        out_specs=pl.BlockSpec((_BN, m), lambda i: (i, 0)),
        out_shape=jax.ShapeDtypeStruct((n, m), jnp.float32),
        compiler_params=pltpu.CompilerParams(
            dimension_semantics=("parallel",)),
    )(I, a_perm, t_perm, tile, kk, l4)


# speedup vs baseline: 1.0616x; 1.0157x over previous
import numpy as np

import jax
import jax.numpy as jnp
from jax.experimental import pallas as pl
from jax.experimental.pallas import tpu as pltpu

_C = 64
_K = 16
_DEPTH = 4
_NODES = _K - 1
_BN = 1024
_HALF = _BN // 2


def _halut_block_kernel(i_ref, a_ref, t_ref, perm_ref, tile_ref, kk_ref,
                        l_ref, o_ref, a_scr, l_scr):
    f32 = jnp.float32

    @pl.when(pl.program_id(0) == 0)
    def _prep():
        a_scr[...] = jnp.dot(a_ref[...].astype(jnp.bfloat16), perm_ref[...],
                             preferred_element_type=f32).astype(jnp.bfloat16)
        l_scr[...] = l_ref[...].astype(jnp.bfloat16)

    ib = i_ref[...].astype(jnp.bfloat16)
    ia0 = jnp.dot(ib[:_HALF], a_scr[...], preferred_element_type=f32)
    ia1 = jnp.dot(ib[_HALF:], a_scr[...], preferred_element_type=f32)

    hs = []
    for lvl in range(_DEPTH):
        sl = slice(lvl * _C, (lvl + 1) * _C)
        h = jnp.concatenate([ia0[:, sl], ia1[:, sl]], axis=1)
        hs.append(h.astype(jnp.bfloat16).astype(f32))

    d = []
    for i in range(_NODES):
        lvl = (i + 1).bit_length() - 1
        di = jnp.tanh(hs[lvl] - t_ref[i, :][None, :])
        d.append(di.astype(jnp.bfloat16).astype(f32))

    p1 = [-d[0], d[0]]
    p2, p3, bs = [], [], []
    for j in range(4):
        b0, b1 = (j >> 1) & 1, j & 1
        n1 = 1 + b0
        p2.append(p1[b0] + (d[n1] if b1 else -d[n1]))
    for j in range(8):
        b0, b1, b2 = (j >> 2) & 1, (j >> 1) & 1, j & 1
        n2 = 3 + 2 * b0 + b1
        p3.append(p2[j >> 1] + (d[n2] if b2 else -d[n2]))
    for k in range(_K):
        b0, b1, b2 = (k >> 3) & 1, (k >> 2) & 1, (k >> 1) & 1
        n3 = 7 + 4 * b0 + 2 * b1 + b2
        bs.append(p3[k >> 1] + (d[n3] if (k & 1) else -d[n3]))

    best_v = bs[0]
    best_k = jnp.zeros(bs[0].shape, dtype=f32)
    for k in range(1, _K):
        upd = bs[k] > best_v
        best_v = jnp.where(upd, bs[k], best_v)
        best_k = jnp.where(upd, f32(k), best_k)

    codes = best_k.astype(jnp.bfloat16)

    for half, codes_h in ((0, codes[:, :_C]), (1, codes[:, _C:])):
        codes_t = jnp.dot(codes_h, tile_ref[...], preferred_element_type=f32)
        e = (codes_t == kk_ref[0:1, :]).astype(jnp.bfloat16)
        out = jax.lax.dot_general(
            e, l_scr[...], (((1,), (1,)), ((), ())),
            preferred_element_type=f32)
        o_ref[half * _HALF:(half + 1) * _HALF] = out


def kernel(I, T, L, S, B, A):
    del S, B
    n, dim = I.shape
    m = L.shape[0]
    cd = _C * _DEPTH
    t15 = T.reshape(_C, _NODES).T
    t_perm = jnp.pad(jnp.concatenate([t15, t15], axis=1), ((0, 1), (0, 0)))
    pmat = np.zeros((cd, cd), dtype=np.float32)
    for c in range(_C):
        for lv in range(_DEPTH):
            pmat[c * _DEPTH + lv, lv * _C + c] = 1.0
    perm = jnp.asarray(pmat, dtype=jnp.bfloat16)
    tile = jnp.asarray(np.repeat(np.eye(_C, dtype=np.float32), _K, axis=1),
                       dtype=jnp.bfloat16)
    kk = jnp.asarray(np.broadcast_to(
        (np.arange(_K * _C) % _K).astype(np.float32), (8, _K * _C)))

    return pl.pallas_call(
        _halut_block_kernel,
        grid=(n // _BN,),
        in_specs=[
            pl.BlockSpec((_BN, dim), lambda i: (i, 0)),
            pl.BlockSpec((dim, cd), lambda i: (0, 0)),
            pl.BlockSpec((_K, 2 * _C), lambda i: (0, 0)),
            pl.BlockSpec((cd, cd), lambda i: (0, 0)),
            pl.BlockSpec((_C, _K * _C), lambda i: (0, 0)),
            pl.BlockSpec((8, _K * _C), lambda i: (0, 0)),
            pl.BlockSpec((m, _K * _C), lambda i: (0, 0)),
        ],
        out_specs=pl.BlockSpec((_BN, m), lambda i: (i, 0)),
        out_shape=jax.ShapeDtypeStruct((n, m), jnp.float32),
        scratch_shapes=[
            pltpu.VMEM((dim, cd), jnp.bfloat16),
            pltpu.VMEM((m, _K * _C), jnp.bfloat16),
        ],
        compiler_params=pltpu.CompilerParams(
            dimension_semantics=("arbitrary",)),
    )(I, A, t_perm, perm, tile, kk, L.reshape(m, _C * _K))
